# Initial kernel scaffold; baseline (speedup 1.0000x reference)
import jax
import jax.numpy as jnp
from jax.experimental import pallas as pl


def kernel(x_nchw, w_taps):
    raise NotImplementedError("write your pallas kernel here")


# speedup vs baseline: 1.1040x; 1.1040x over previous
from functools import partial

import jax
import jax.numpy as jnp
from jax.experimental import pallas as pl
from jax.experimental.pallas import tpu as pltpu


def _eca_fused_kernel(x_ref, w_ref, o_ref, *, k, inv_hw):
    x = x_ref[...]
    c = x.shape[0]
    pad = (k - 1) // 2

    s = jnp.sum(x, axis=-1, keepdims=True)

    zpad = jnp.zeros((pad, 1), jnp.float32)
    sp = jnp.concatenate([zpad, s, zpad], axis=0)
    y = w_ref[0] * sp[0:c, :]
    for j in range(1, k):
        y = y + w_ref[j] * sp[j:j + c, :]
    gate = jax.nn.sigmoid(y * inv_hw)

    o_ref[...] = x * gate


def kernel(x_nchw, w_taps):
    n, c, h, w = x_nchw.shape
    hw = h * w
    k = w_taps.shape[0]

    x2d = x_nchw.reshape(n * c, hw)

    out2d = pl.pallas_call(
        partial(_eca_fused_kernel, k=k, inv_hw=1.0 / hw),
        out_shape=jax.ShapeDtypeStruct((n * c, hw), x2d.dtype),
        grid=(n,),
        in_specs=[
            pl.BlockSpec((c, hw), lambda i: (i, 0)),
            pl.BlockSpec(memory_space=pltpu.MemorySpace.SMEM),
        ],
        out_specs=pl.BlockSpec((c, hw), lambda i: (i, 0)),
        compiler_params=pltpu.CompilerParams(
            dimension_semantics=("parallel",),
            vmem_limit_bytes=80 * 1024 * 1024,
        ),
    )(x2d, w_taps.astype(jnp.float32))

    return out2d.reshape(n, c, h, w)


# speedup vs baseline: 1.2117x; 1.0976x over previous
from functools import partial

import jax
import jax.numpy as jnp
from jax.experimental import pallas as pl
from jax.experimental.pallas import tpu as pltpu


def _gate_kernel(x_ref, w_ref, g_ref, *, k, inv_hw):
    c = x_ref.shape[1]
    pad = (k - 1) // 2

    s = jnp.sum(x_ref[...], axis=(2, 3), keepdims=True)

    zpad = jnp.zeros((1, pad, 1, 1), jnp.float32)
    sp = jnp.concatenate([zpad, s, zpad], axis=1)
    y = w_ref[0] * sp[:, 0:c]
    for j in range(1, k):
        y = y + w_ref[j] * sp[:, j:j + c]
    g_ref[...] = jax.nn.sigmoid(y * inv_hw)


def _scale_kernel(x_ref, g_ref, o_ref):
    o_ref[...] = x_ref[...] * g_ref[...]


def kernel(x_nchw, w_taps):
    n, c, h, w = x_nchw.shape
    k = w_taps.shape[0]

    gates = pl.pallas_call(
        partial(_gate_kernel, k=k, inv_hw=1.0 / (h * w)),
        out_shape=jax.ShapeDtypeStruct((n, c, 1, 1), jnp.float32),
        grid=(n,),
        in_specs=[
            pl.BlockSpec((1, c, h, w), lambda i: (i, 0, 0, 0)),
            pl.BlockSpec(memory_space=pltpu.MemorySpace.SMEM),
        ],
        out_specs=pl.BlockSpec((1, c, 1, 1), lambda i: (i, 0, 0, 0)),
        compiler_params=pltpu.CompilerParams(
            dimension_semantics=("parallel",),
            vmem_limit_bytes=60 * 1024 * 1024,
        ),
    )(x_nchw, w_taps.astype(jnp.float32))

    ct = c // 2
    out = pl.pallas_call(
        _scale_kernel,
        out_shape=jax.ShapeDtypeStruct((n, c, h, w), x_nchw.dtype),
        grid=(n, c // ct),
        in_specs=[
            pl.BlockSpec((1, ct, h, w), lambda i, j: (i, j, 0, 0)),
            pl.BlockSpec((1, ct, 1, 1), lambda i, j: (i, j, 0, 0)),
        ],
        out_specs=pl.BlockSpec((1, ct, h, w), lambda i, j: (i, j, 0, 0)),
        compiler_params=pltpu.CompilerParams(
            dimension_semantics=("parallel", "parallel"),
            vmem_limit_bytes=60 * 1024 * 1024,
        ),
    )(x_nchw, gates)

    return out


# speedup vs baseline: 1.3972x; 1.1531x over previous
from functools import partial

import jax
import jax.numpy as jnp
from jax.experimental import pallas as pl
from jax.experimental.pallas import tpu as pltpu


def _eca_fused_kernel(x_ref, w_ref, o_ref, out_sem, *, k, inv_hw):
    i = pl.program_id(0)
    c = x_ref.shape[1]
    pad = (k - 1) // 2

    s = jnp.sum(x_ref[...], axis=(2, 3), keepdims=True)

    zpad = jnp.zeros((1, pad, 1, 1), jnp.float32)
    sp = jnp.concatenate([zpad, s, zpad], axis=1)
    y = w_ref[0] * sp[:, 0:c]
    for j in range(1, k):
        y = y + w_ref[j] * sp[:, j:j + c]
    gate = jax.nn.sigmoid(y * inv_hw)

    x_ref[...] = x_ref[...] * gate
    copy = pltpu.make_async_copy(x_ref, o_ref.at[pl.ds(i, 1)], out_sem)
    copy.start()
    copy.wait()


def kernel(x_nchw, w_taps):
    n, c, h, w = x_nchw.shape
    k = w_taps.shape[0]

    out = pl.pallas_call(
        partial(_eca_fused_kernel, k=k, inv_hw=1.0 / (h * w)),
        out_shape=jax.ShapeDtypeStruct((n, c, h, w), x_nchw.dtype),
        grid=(n,),
        in_specs=[
            pl.BlockSpec((1, c, h, w), lambda i: (i, 0, 0, 0)),
            pl.BlockSpec(memory_space=pltpu.MemorySpace.SMEM),
        ],
        out_specs=pl.BlockSpec(memory_space=pltpu.MemorySpace.HBM),
        scratch_shapes=[pltpu.SemaphoreType.DMA],
        compiler_params=pltpu.CompilerParams(
            dimension_semantics=("parallel",),
            vmem_limit_bytes=56 * 1024 * 1024,
        ),
    )(x_nchw, w_taps.astype(jnp.float32))

    return out


# speedup vs baseline: 1.4053x; 1.0058x over previous
from functools import partial

import jax
import jax.numpy as jnp
from jax.experimental import pallas as pl
from jax.experimental.pallas import tpu as pltpu


def _eca_fused_kernel(x_hbm, w_ref, o_hbm, x_buf, in_sem, out_sem, *, k, inv_hw):
    j = pl.program_id(1)
    nb = pl.num_programs(1)
    b = pl.program_id(0) * nb + j
    slot = jax.lax.rem(j, 2)
    nxt = jax.lax.rem(j + 1, 2)
    c = x_buf.shape[2]
    pad = (k - 1) // 2

    def in_copy(buf_slot, batch):
        return pltpu.make_async_copy(
            x_hbm.at[pl.ds(batch, 1)], x_buf.at[buf_slot], in_sem.at[buf_slot])

    def out_copy(buf_slot, batch):
        return pltpu.make_async_copy(
            x_buf.at[buf_slot], o_hbm.at[pl.ds(batch, 1)], out_sem.at[buf_slot])

    @pl.when(j == 0)
    def _():
        in_copy(slot, b).start()

    @pl.when(j + 1 < nb)
    def _():
        @pl.when(j >= 1)
        def _():
            out_copy(nxt, b - 1).wait()
        in_copy(nxt, b + 1).start()

    in_copy(slot, b).wait()

    xs = x_buf[slot]
    s = jnp.sum(xs, axis=(2, 3), keepdims=True)

    zpad = jnp.zeros((1, pad, 1, 1), jnp.float32)
    sp = jnp.concatenate([zpad, s, zpad], axis=1)
    y = w_ref[0] * sp[:, 0:c]
    for t in range(1, k):
        y = y + w_ref[t] * sp[:, t:t + c]
    gate = jax.nn.sigmoid(y * inv_hw)

    x_buf[slot] = xs * gate
    out_copy(slot, b).start()

    @pl.when(j == nb - 1)
    def _():
        @pl.when(j >= 1)
        def _():
            out_copy(nxt, b - 1).wait()
        out_copy(slot, b).wait()


def kernel(x_nchw, w_taps):
    n, c, h, w = x_nchw.shape
    k = w_taps.shape[0]
    cores = 2 if n % 2 == 0 else 1

    out = pl.pallas_call(
        partial(_eca_fused_kernel, k=k, inv_hw=1.0 / (h * w)),
        out_shape=jax.ShapeDtypeStruct((n, c, h, w), x_nchw.dtype),
        grid=(cores, n // cores),
        in_specs=[
            pl.BlockSpec(memory_space=pltpu.MemorySpace.HBM),
            pl.BlockSpec(memory_space=pltpu.MemorySpace.SMEM),
        ],
        out_specs=pl.BlockSpec(memory_space=pltpu.MemorySpace.HBM),
        scratch_shapes=[
            pltpu.VMEM((2, 1, c, h, w), jnp.float32),
            pltpu.SemaphoreType.DMA((2,)),
            pltpu.SemaphoreType.DMA((2,)),
        ],
        compiler_params=pltpu.CompilerParams(
            dimension_semantics=("parallel", "arbitrary"),
            vmem_limit_bytes=56 * 1024 * 1024,
        ),
    )(x_nchw, w_taps.astype(jnp.float32))

    return out
